# baseline (speedup 1.0000x reference)
import functools

import jax
import jax.numpy as jnp
from jax import lax
from jax.experimental import pallas as pl
from jax.experimental.pallas import tpu as pltpu
from jax.experimental.pallas import tpu_sc as plsc

N = 10000
NP = 10240
E = 320000
D_IN = 128
D_HID = 64

NC, NS = 2, 16
NW = NC * NS
CHUNK = 128
JPT = 80
EPT = JPT * CHUNK
EPAD = NW * EPT
TRASH = N
RPT = NP // NS

_MESH = plsc.VectorSubcoreMesh(core_axis_name="c", subcore_axis_name="s")


@functools.partial(
    pl.kernel,
    out_type=jax.ShapeDtypeStruct((NC, NP), jnp.float32),
    mesh=_MESH,
    scratch_types=[
        pltpu.VMEM_SHARED((NP,), jnp.float32),
        pltpu.VMEM((JPT, CHUNK), jnp.int32),
        pltpu.VMEM((CHUNK,), jnp.float32),
        pltpu.VMEM((RPT,), jnp.float32),
        pltpu.SemaphoreType.DMA,
    ],
    compiler_params=pltpu.CompilerParams(use_tc_tiling_on_sc=False),
)
def _deg_kernel(ei_hbm, out_hbm, acc, dst_idx, ones_v, zeros_v, sem):
    cid = lax.axis_index("c")
    sid = lax.axis_index("s")
    wid = cid * NS + sid

    for i in range(RPT // 16):
        zeros_v[pl.ds(16 * i, 16)] = jnp.zeros((16,), jnp.float32)
    for i in range(CHUNK // 16):
        ones_v[pl.ds(16 * i, 16)] = jnp.ones((16,), jnp.float32)

    pltpu.sync_copy(ei_hbm.at[1, pl.ds(wid * JPT, JPT)], dst_idx)
    pltpu.sync_copy(zeros_v, acc.at[pl.ds(sid * RPT, RPT)])
    plsc.subcore_barrier()

    def fire(j, carry):
        pltpu.async_copy(ones_v, acc.at[dst_idx.at[j]], sem, add=True)
        return carry

    lax.fori_loop(0, JPT, fire, 0)

    def drain(j, carry):
        pltpu.make_async_copy(ones_v, acc.at[dst_idx.at[j]], sem).wait()
        return carry

    lax.fori_loop(0, JPT, drain, 0)
    plsc.subcore_barrier()
    pltpu.sync_copy(acc.at[pl.ds(sid * RPT, RPT)],
                    out_hbm.at[cid, pl.ds(sid * RPT, RPT)])


@functools.partial(
    pl.kernel,
    out_type=jax.ShapeDtypeStruct((NC, NP, D_HID), jnp.float32),
    mesh=_MESH,
    scratch_types=[
        pltpu.VMEM_SHARED((NP, D_HID), jnp.float32),
        pltpu.VMEM_SHARED((NP, D_HID), jnp.float32),
        pltpu.VMEM((JPT // 2, CHUNK), jnp.int32),
        pltpu.VMEM((JPT // 2, CHUNK), jnp.int32),
        pltpu.VMEM((CHUNK, D_HID), jnp.float32),
        pltpu.VMEM((CHUNK, D_HID), jnp.float32),
        pltpu.VMEM((CHUNK, D_HID), jnp.float32),
        pltpu.VMEM((CHUNK, D_HID), jnp.float32),
        pltpu.SemaphoreType.DMA,
        pltpu.SemaphoreType.DMA,
        pltpu.SemaphoreType.DMA,
        pltpu.SemaphoreType.DMA,
    ],
    compiler_params=pltpu.CompilerParams(use_tc_tiling_on_sc=False),
)
def _agg_kernel(y_hbm, ei_hbm, out_hbm, acc, ytab, src_idx, dst_idx,
                rows0, rows1, rows2, rows3, sem0, sem1, sem2, sem3):
    cid = lax.axis_index("c")
    sid = lax.axis_index("s")
    wid = cid * NS + sid
    rows = (rows0, rows1, rows2, rows3)
    sems = (sem0, sem1, sem2, sem3)

    r0 = sid * RPT
    pltpu.sync_copy(y_hbm.at[pl.ds(r0, RPT)], ytab.at[pl.ds(r0, RPT)])
    pltpu.sync_copy(ytab.at[pl.ds(r0, RPT)], acc.at[pl.ds(r0, RPT)])
    plsc.subcore_barrier()

    def gather(j, k):
        pltpu.async_copy(ytab.at[src_idx.at[j]], rows[k], sems[k])

    def gather_wait(j, k):
        pltpu.make_async_copy(ytab.at[src_idx.at[j]], rows[k], sems[k]).wait()

    def scatter(j, k):
        pltpu.async_copy(rows[k], acc.at[dst_idx.at[j]], sems[k], add=True)

    def scatter_wait(j, k):
        pltpu.make_async_copy(rows[k], acc.at[dst_idx.at[j]], sems[k]).wait()

    JH = JPT // 2
    for phase in range(2):
        base_c = wid * JPT + phase * JH
        pltpu.sync_copy(ei_hbm.at[0, pl.ds(base_c, JH)], src_idx)
        pltpu.sync_copy(ei_hbm.at[1, pl.ds(base_c, JH)], dst_idx)

        gather(0, 0)
        gather(1, 1)
        gather_wait(0, 0)
        scatter(0, 0)
        gather(2, 2)
        gather_wait(1, 1)
        scatter(1, 1)
        gather(3, 3)

        def body(jj, carry):
            base = 4 * jj + 2
            for t in range(4):
                j = base + t
                k = (2 + t) % 4
                kn = t
                gather_wait(j, k)
                scatter(j, k)
                scatter_wait(j - 2, kn)
                gather(j + 2, kn)
            return carry

        lax.fori_loop(0, (JH - 4) // 4, body, 0)
        gather_wait(JH - 2, 2)
        scatter(JH - 2, 2)
        scatter_wait(JH - 4, 0)
        gather_wait(JH - 1, 3)
        scatter(JH - 1, 3)
        scatter_wait(JH - 3, 1)
        scatter_wait(JH - 2, 2)
        scatter_wait(JH - 1, 3)
    plsc.subcore_barrier()
    pltpu.sync_copy(acc.at[pl.ds(r0, RPT)], out_hbm.at[cid, pl.ds(r0, RPT)])


_BS = 1024


def _tc_scale_matmul(x_ref, w_ref, deg_ref, y_ref, dinv_ref):
    deg = 1.0 + deg_ref[0] + deg_ref[1]
    dinv = lax.rsqrt(deg)
    y_ref[...] = dinv * jnp.dot(x_ref[...], w_ref[...],
                                preferred_element_type=jnp.float32)
    dinv_ref[...] = dinv


def _tc_mid(acc_ref, y_ref, dinv_ref, bias_ref, w_ref, out_ref):
    s = acc_ref[0] + acc_ref[1] - y_ref[...]
    h = jnp.maximum(dinv_ref[...] * s + bias_ref[...], 0.0)
    out_ref[...] = dinv_ref[...] * jnp.dot(h, w_ref[...],
                                           preferred_element_type=jnp.float32)


def _tc_out(acc_ref, y_ref, dinv_ref, bias_ref, w_ref, bout_ref, out_ref):
    s = acc_ref[0] + acc_ref[1] - y_ref[...]
    h = jnp.maximum(dinv_ref[...] * s + bias_ref[...], 0.0)
    out_ref[...] = jnp.dot(h, w_ref[...],
                           preferred_element_type=jnp.float32) + bout_ref[...]


_scale_matmul = pl.pallas_call(
    _tc_scale_matmul,
    out_shape=[jax.ShapeDtypeStruct((NP, D_HID), jnp.float32),
               jax.ShapeDtypeStruct((NP, 1), jnp.float32)],
)

_mid = pl.pallas_call(
    _tc_mid,
    out_shape=jax.ShapeDtypeStruct((NP, D_HID), jnp.float32),
)

_out = pl.pallas_call(
    _tc_out,
    out_shape=jax.ShapeDtypeStruct((NP, 1), jnp.float32),
)


def kernel(x, edge_index, W1, b1, W2, b2, Wout, bout):
    pad = EPAD - E
    eip = jnp.pad(edge_index, ((0, 0), (0, pad)),
                  constant_values=TRASH).reshape(2, NW * JPT, CHUNK)
    xp = jnp.pad(x, ((0, NP - N), (0, 0)))

    degp = _deg_kernel(eip).reshape(NC, NP, 1)

    y1, dinv = _scale_matmul(xp, W1, degp)
    acc1 = _agg_kernel(y1, eip)
    y2 = _mid(acc1, y1, dinv, b1.reshape(1, D_HID), W2)
    acc2 = _agg_kernel(y2, eip)
    out = _out(acc2, y2, dinv, b2.reshape(1, D_HID), Wout, bout.reshape(1, 1))
    return out.reshape(NP)[:N]


# speedup vs baseline: 2.1672x; 2.1672x over previous
import functools

import jax
import jax.numpy as jnp
from jax import lax
from jax.experimental import pallas as pl
from jax.experimental.pallas import tpu as pltpu
from jax.experimental.pallas import tpu_sc as plsc

N = 10000
NP = 10240
E = 320000
D_IN = 128
D_HID = 64

NC, NS = 2, 16
NW = NC * NS
CHUNK = 128
JPT = 80
EPT = JPT * CHUNK
EPAD = NW * EPT
TRASH = N
RPT = NP // NS

_MESH = plsc.VectorSubcoreMesh(core_axis_name="c", subcore_axis_name="s")


@functools.partial(
    pl.kernel,
    out_type=jax.ShapeDtypeStruct((NC, NP), jnp.float32),
    mesh=_MESH,
    scratch_types=[
        pltpu.VMEM_SHARED((NP,), jnp.float32),
        pltpu.VMEM((JPT, CHUNK), jnp.int32),
        pltpu.VMEM((CHUNK,), jnp.float32),
        pltpu.VMEM((RPT,), jnp.float32),
        pltpu.SemaphoreType.DMA,
    ],
    compiler_params=pltpu.CompilerParams(use_tc_tiling_on_sc=False),
)
def _deg_kernel(ei_hbm, out_hbm, acc, dst_idx, ones_v, zeros_v, sem):
    cid = lax.axis_index("c")
    sid = lax.axis_index("s")
    wid = cid * NS + sid

    for i in range(RPT // 16):
        zeros_v[pl.ds(16 * i, 16)] = jnp.zeros((16,), jnp.float32)
    for i in range(CHUNK // 16):
        ones_v[pl.ds(16 * i, 16)] = jnp.ones((16,), jnp.float32)

    pltpu.sync_copy(ei_hbm.at[1, pl.ds(wid * JPT, JPT)], dst_idx)
    pltpu.sync_copy(zeros_v, acc.at[pl.ds(sid * RPT, RPT)])
    plsc.subcore_barrier()

    def fire(j, carry):
        pltpu.async_copy(ones_v, acc.at[dst_idx.at[j]], sem, add=True)
        return carry

    lax.fori_loop(0, JPT, fire, 0)

    def drain(j, carry):
        pltpu.make_async_copy(ones_v, acc.at[dst_idx.at[j]], sem).wait()
        return carry

    lax.fori_loop(0, JPT, drain, 0)
    plsc.subcore_barrier()
    pltpu.sync_copy(acc.at[pl.ds(sid * RPT, RPT)],
                    out_hbm.at[cid, pl.ds(sid * RPT, RPT)])


@functools.partial(
    pl.kernel,
    out_type=jax.ShapeDtypeStruct((NC, NP, D_HID), jnp.float32),
    mesh=_MESH,
    scratch_types=[
        pltpu.VMEM_SHARED((NP, D_HID), jnp.float32),
        pltpu.VMEM_SHARED((NP, D_HID), jnp.float32),
        pltpu.VMEM((JPT // 2, CHUNK), jnp.int32),
        pltpu.VMEM((JPT // 2, CHUNK), jnp.int32),
        pltpu.VMEM((CHUNK, D_HID), jnp.float32),
        pltpu.VMEM((CHUNK, D_HID), jnp.float32),
        pltpu.VMEM((CHUNK, D_HID), jnp.float32),
        pltpu.VMEM((CHUNK, D_HID), jnp.float32),
        pltpu.SemaphoreType.DMA,
        pltpu.SemaphoreType.DMA,
        pltpu.SemaphoreType.DMA,
        pltpu.SemaphoreType.DMA,
    ],
    compiler_params=pltpu.CompilerParams(use_tc_tiling_on_sc=False),
)
def _agg_kernel(y_hbm, ei_hbm, out_hbm, acc, ytab, src_idx, dst_idx,
                rows0, rows1, rows2, rows3, sem0, sem1, sem2, sem3):
    cid = lax.axis_index("c")
    sid = lax.axis_index("s")
    wid = cid * NS + sid
    rows = (rows0, rows1, rows2, rows3)
    sems = (sem0, sem1, sem2, sem3)

    r0 = sid * RPT
    pltpu.sync_copy(y_hbm.at[pl.ds(r0, RPT)], acc.at[pl.ds(r0, RPT)])
    pltpu.sync_copy(y_hbm.at[pl.ds(r0, RPT)], ytab.at[pl.ds(r0, RPT)])
    plsc.subcore_barrier()

    def gather(j, k):
        pltpu.async_copy(ytab.at[src_idx.at[j]], rows[k], sems[k])

    def gather_wait(j, k):
        pltpu.make_async_copy(ytab.at[src_idx.at[j]], rows[k], sems[k]).wait()

    def scatter(j, k):
        pltpu.async_copy(rows[k], acc.at[dst_idx.at[j]], sems[k], add=True)

    def scatter_wait(j, k):
        pltpu.make_async_copy(rows[k], acc.at[dst_idx.at[j]], sems[k]).wait()

    JH = JPT // 2
    for phase in range(2):
        base_c = wid * JPT + phase * JH
        pltpu.sync_copy(ei_hbm.at[0, pl.ds(base_c, JH)], src_idx)
        pltpu.sync_copy(ei_hbm.at[1, pl.ds(base_c, JH)], dst_idx)

        gather(0, 0)
        gather(1, 1)
        gather_wait(0, 0)
        scatter(0, 0)
        gather(2, 2)
        gather_wait(1, 1)
        scatter(1, 1)
        gather(3, 3)

        def body(jj, carry):
            base = 4 * jj + 2
            for t in range(4):
                j = base + t
                k = (2 + t) % 4
                kn = t
                gather_wait(j, k)
                scatter(j, k)
                scatter_wait(j - 2, kn)
                gather(j + 2, kn)
            return carry

        lax.fori_loop(0, (JH - 4) // 4, body, 0)
        gather_wait(JH - 2, 2)
        scatter(JH - 2, 2)
        scatter_wait(JH - 4, 0)
        gather_wait(JH - 1, 3)
        scatter(JH - 1, 3)
        scatter_wait(JH - 3, 1)
        scatter_wait(JH - 2, 2)
        scatter_wait(JH - 1, 3)
    plsc.subcore_barrier()
    pltpu.sync_copy(acc.at[pl.ds(r0, RPT)], out_hbm.at[cid, pl.ds(r0, RPT)])


_BS = 1024


def _tc_scale_matmul(x_ref, w_ref, deg_ref, y_ref, dinv_ref):
    deg = 1.0 + deg_ref[0] + deg_ref[1]
    dinv = lax.rsqrt(deg)
    y_ref[...] = dinv * jnp.dot(x_ref[...], w_ref[...],
                                preferred_element_type=jnp.float32)
    dinv_ref[...] = dinv


def _tc_mid(acc_ref, y_ref, dinv_ref, bias_ref, w_ref, out_ref):
    s = acc_ref[0] + acc_ref[1] - y_ref[...]
    h = jnp.maximum(dinv_ref[...] * s + bias_ref[...], 0.0)
    out_ref[...] = dinv_ref[...] * jnp.dot(h, w_ref[...],
                                           preferred_element_type=jnp.float32)


def _tc_out(acc_ref, y_ref, dinv_ref, bias_ref, w_ref, bout_ref, out_ref):
    s = acc_ref[0] + acc_ref[1] - y_ref[...]
    h = jnp.maximum(dinv_ref[...] * s + bias_ref[...], 0.0)
    out_ref[...] = jnp.dot(h, w_ref[...],
                           preferred_element_type=jnp.float32) + bout_ref[...]


_scale_matmul = pl.pallas_call(
    _tc_scale_matmul,
    out_shape=[jax.ShapeDtypeStruct((NP, D_HID), jnp.float32),
               jax.ShapeDtypeStruct((NP, 1), jnp.float32)],
)

_mid = pl.pallas_call(
    _tc_mid,
    out_shape=jax.ShapeDtypeStruct((NP, D_HID), jnp.float32),
)

_out = pl.pallas_call(
    _tc_out,
    out_shape=jax.ShapeDtypeStruct((NP, 1), jnp.float32),
)


def kernel(x, edge_index, W1, b1, W2, b2, Wout, bout):
    pad = EPAD - E
    eip = jnp.pad(edge_index, ((0, 0), (0, pad)),
                  constant_values=TRASH).reshape(2, NW * JPT, CHUNK)
    xp = jnp.pad(x, ((0, NP - N), (0, 0)))

    degp = _deg_kernel(eip).reshape(NC, NP, 1)

    y1, dinv = _scale_matmul(xp, W1, degp)
    acc1 = _agg_kernel(y1, eip)
    y2 = _mid(acc1, y1, dinv, b1.reshape(1, D_HID), W2)
    acc2 = _agg_kernel(y2, eip)
    out = _out(acc2, y2, dinv, b2.reshape(1, D_HID), Wout, bout.reshape(1, 1))
    return out.reshape(NP)[:N]


# speedup vs baseline: 2.1757x; 1.0039x over previous
import functools

import jax
import jax.numpy as jnp
from jax import lax
from jax.experimental import pallas as pl
from jax.experimental.pallas import tpu as pltpu
from jax.experimental.pallas import tpu_sc as plsc

N = 10000
NP = 10240
E = 320000
D_IN = 128
D_HID = 64

NC, NS = 2, 16
NW = NC * NS
CHUNK = 128
JPT = 80
EPT = JPT * CHUNK
EPAD = NW * EPT
TRASH = N
RPT = NP // NS

_MESH = plsc.VectorSubcoreMesh(core_axis_name="c", subcore_axis_name="s")


@functools.partial(
    pl.kernel,
    out_type=jax.ShapeDtypeStruct((NC, NP), jnp.float32),
    mesh=_MESH,
    scratch_types=[
        pltpu.VMEM_SHARED((NP,), jnp.float32),
        pltpu.VMEM((JPT, CHUNK), jnp.int32),
        pltpu.VMEM((CHUNK,), jnp.float32),
        pltpu.VMEM((RPT,), jnp.float32),
        pltpu.SemaphoreType.DMA,
    ],
    compiler_params=pltpu.CompilerParams(use_tc_tiling_on_sc=False),
)
def _deg_kernel(ei_hbm, out_hbm, acc, dst_idx, ones_v, zeros_v, sem):
    cid = lax.axis_index("c")
    sid = lax.axis_index("s")
    wid = cid * NS + sid

    for i in range(RPT // 16):
        zeros_v[pl.ds(16 * i, 16)] = jnp.zeros((16,), jnp.float32)
    for i in range(CHUNK // 16):
        ones_v[pl.ds(16 * i, 16)] = jnp.ones((16,), jnp.float32)

    pltpu.sync_copy(ei_hbm.at[1, pl.ds(wid * JPT, JPT)], dst_idx)
    pltpu.sync_copy(zeros_v, acc.at[pl.ds(sid * RPT, RPT)])
    plsc.subcore_barrier()

    def fire(j, carry):
        pltpu.async_copy(ones_v, acc.at[dst_idx.at[j]], sem, add=True)
        return carry

    lax.fori_loop(0, JPT, fire, 0)

    def drain(j, carry):
        pltpu.make_async_copy(ones_v, acc.at[dst_idx.at[j]], sem).wait()
        return carry

    lax.fori_loop(0, JPT, drain, 0)
    plsc.subcore_barrier()
    pltpu.sync_copy(acc.at[pl.ds(sid * RPT, RPT)],
                    out_hbm.at[cid, pl.ds(sid * RPT, RPT)])


@functools.partial(
    pl.kernel,
    out_type=jax.ShapeDtypeStruct((NC, NP, D_HID), jnp.float32),
    mesh=_MESH,
    scratch_types=[
        pltpu.VMEM_SHARED((NP, D_HID), jnp.float32),
        pltpu.VMEM_SHARED((NP, D_HID), jnp.float32),
        pltpu.VMEM((JPT // 2, CHUNK), jnp.int32),
        pltpu.VMEM((JPT // 2, CHUNK), jnp.int32),
        pltpu.VMEM((CHUNK, D_HID), jnp.float32),
        pltpu.VMEM((CHUNK, D_HID), jnp.float32),
        pltpu.VMEM((CHUNK, D_HID), jnp.float32),
        pltpu.VMEM((CHUNK, D_HID), jnp.float32),
        pltpu.SemaphoreType.DMA,
        pltpu.SemaphoreType.DMA,
        pltpu.SemaphoreType.DMA,
        pltpu.SemaphoreType.DMA,
    ],
    compiler_params=pltpu.CompilerParams(use_tc_tiling_on_sc=False),
)
def _agg_kernel(y_hbm, ei_hbm, out_hbm, acc, ytab, src_idx, dst_idx,
                rows0, rows1, rows2, rows3, sem0, sem1, sem2, sem3):
    cid = lax.axis_index("c")
    sid = lax.axis_index("s")
    wid = cid * NS + sid
    rows = (rows0, rows1, rows2, rows3)
    sems = (sem0, sem1, sem2, sem3)

    r0 = sid * RPT
    pltpu.sync_copy(y_hbm.at[pl.ds(r0, RPT)], acc.at[pl.ds(r0, RPT)])
    pltpu.sync_copy(y_hbm.at[pl.ds(r0, RPT)], ytab.at[pl.ds(r0, RPT)])
    plsc.subcore_barrier()

    def gather(j, k):
        pltpu.async_copy(ytab.at[src_idx.at[j]], rows[k], sems[k])

    def gather_wait(j, k):
        pltpu.make_async_copy(ytab.at[src_idx.at[j]], rows[k], sems[k]).wait()

    def scatter(j, k):
        pltpu.async_copy(rows[k], acc.at[dst_idx.at[j]], sems[k], add=True)

    def scatter_wait(j, k):
        pltpu.make_async_copy(rows[k], acc.at[dst_idx.at[j]], sems[k]).wait()

    JH = JPT // 2
    for phase in range(2):
        base_c = wid * JPT + phase * JH
        pltpu.sync_copy(ei_hbm.at[0, pl.ds(base_c, JH)], src_idx)
        pltpu.sync_copy(ei_hbm.at[1, pl.ds(base_c, JH)], dst_idx)

        gather(0, 0)
        gather(1, 1)
        gather_wait(0, 0)
        scatter(0, 0)
        gather(2, 2)
        gather_wait(1, 1)
        scatter(1, 1)
        gather(3, 3)

        def body(jj, carry):
            base = 4 * jj + 2
            for t in range(4):
                j = base + t
                k = (2 + t) % 4
                kn = t
                gather_wait(j, k)
                scatter(j, k)
                scatter_wait(j - 2, kn)
                gather(j + 2, kn)
            return carry

        lax.fori_loop(0, (JH - 4) // 4, body, 0)
        gather_wait(JH - 2, 2)
        scatter(JH - 2, 2)
        scatter_wait(JH - 4, 0)
        gather_wait(JH - 1, 3)
        scatter(JH - 1, 3)
        scatter_wait(JH - 3, 1)
        scatter_wait(JH - 2, 2)
        scatter_wait(JH - 1, 3)
    plsc.subcore_barrier()
    pltpu.sync_copy(acc.at[pl.ds(r0, RPT)], out_hbm.at[cid, pl.ds(r0, RPT)])


_BS = 1024


def _tc_matmul(x_ref, w_ref, y_ref):
    y_ref[...] = jnp.dot(x_ref[...], w_ref[...],
                         preferred_element_type=jnp.float32)


def _tc_scale(yraw_ref, deg_ref, y_ref, dinv_ref):
    deg = 1.0 + deg_ref[0] + deg_ref[1]
    dinv = lax.rsqrt(deg)
    y_ref[...] = dinv * yraw_ref[...]
    dinv_ref[...] = dinv


def _tc_mid(acc_ref, y_ref, dinv_ref, bias_ref, w_ref, out_ref):
    s = acc_ref[0] + acc_ref[1] - y_ref[...]
    h = jnp.maximum(dinv_ref[...] * s + bias_ref[...], 0.0)
    out_ref[...] = dinv_ref[...] * jnp.dot(h, w_ref[...],
                                           preferred_element_type=jnp.float32)


def _tc_out(acc_ref, y_ref, dinv_ref, bias_ref, w_ref, bout_ref, out_ref):
    s = acc_ref[0] + acc_ref[1] - y_ref[...]
    h = jnp.maximum(dinv_ref[...] * s + bias_ref[...], 0.0)
    out_ref[...] = jnp.dot(h, w_ref[...],
                           preferred_element_type=jnp.float32) + bout_ref[...]


_matmul = pl.pallas_call(
    _tc_matmul,
    out_shape=jax.ShapeDtypeStruct((NP, D_HID), jnp.float32),
)

_scale = pl.pallas_call(
    _tc_scale,
    out_shape=[jax.ShapeDtypeStruct((NP, D_HID), jnp.float32),
               jax.ShapeDtypeStruct((NP, 1), jnp.float32)],
)

_mid = pl.pallas_call(
    _tc_mid,
    out_shape=jax.ShapeDtypeStruct((NP, D_HID), jnp.float32),
)

_out = pl.pallas_call(
    _tc_out,
    out_shape=jax.ShapeDtypeStruct((NP, 1), jnp.float32),
)


def kernel(x, edge_index, W1, b1, W2, b2, Wout, bout):
    pad = EPAD - E
    eip = jnp.pad(edge_index, ((0, 0), (0, pad)),
                  constant_values=TRASH).reshape(2, NW * JPT, CHUNK)
    xp = jnp.pad(x, ((0, NP - N), (0, 0)))

    degp = _deg_kernel(eip).reshape(NC, NP, 1)
    y1_raw = _matmul(xp, W1)

    y1, dinv = _scale(y1_raw, degp)
    acc1 = _agg_kernel(y1, eip)
    y2 = _mid(acc1, y1, dinv, b1.reshape(1, D_HID), W2)
    acc2 = _agg_kernel(y2, eip)
    out = _out(acc2, y2, dinv, b2.reshape(1, D_HID), Wout, bout.reshape(1, 1))
    return out.reshape(NP)[:N]
